# baseline (speedup 1.0000x reference)
import functools

import jax
import jax.numpy as jnp
from jax import lax
from jax.experimental import pallas as pl
from jax.experimental.pallas import tpu as pltpu
from jax.experimental.pallas import tpu_sc as plsc

N = 10000
E = 320000
D = 128
DH = D // 2
BN_EPS = 1e-5

NC = 2
NS = 16

EDGES_PER_TILE = E // NS
CHUNK = 128
NCHUNK = -(-EDGES_PER_TILE // CHUNK)
PAD = NCHUNK * CHUNK - EDGES_PER_TILE
TRASH = N
ACC_ROWS = N + 8
ROWS_PER_TILE = 624
TAIL_ROWS = N - NS * ROWS_PER_TILE
TAIL_BASE = NS * ROWS_PER_TILE


def _sc_scatter(xa, xb, edges):
    mesh = plsc.VectorSubcoreMesh(core_axis_name="c", subcore_axis_name="s")

    @functools.partial(
        pl.kernel,
        mesh=mesh,
        compiler_params=pltpu.CompilerParams(use_tc_tiling_on_sc=False),
        out_type=jax.ShapeDtypeStruct((2, 2, N, DH), jnp.float32),
        scratch_types=[
            pltpu.VMEM((NCHUNK, CHUNK), jnp.int32),
            pltpu.VMEM((NCHUNK, CHUNK), jnp.int32),
            pltpu.VMEM((CHUNK, DH), jnp.float32),
            pltpu.VMEM((CHUNK, DH), jnp.float32),
            pltpu.VMEM((CHUNK, DH), jnp.float32),
            pltpu.VMEM((CHUNK, DH), jnp.float32),
            pltpu.VMEM_SHARED((ACC_ROWS, DH), jnp.float32),
            pltpu.SemaphoreType.DMA,
            pltpu.SemaphoreType.DMA,
            pltpu.SemaphoreType.DMA,
            pltpu.SemaphoreType.DMA,
        ],
    )
    def scatter_kernel(xa_hbm, xb_hbm, edges_hbm, out_hbm,
                       src_v, dst_v, rows0_v, rows1_v, rows2_v, rows3_v,
                       acc_sh, gsem0, gsem1, gsem2, gsem3):
        bufs = [rows0_v, rows1_v, rows2_v, rows3_v]
        gsems = [gsem0, gsem1, gsem2, gsem3]
        cid = lax.axis_index("c")
        sid = lax.axis_index("s")

        pltpu.sync_copy(edges_hbm.at[cid, 0, sid], src_v)
        pltpu.sync_copy(edges_hbm.at[cid, 1, sid], dst_v)

        r0 = sid * ROWS_PER_TILE

        for h, xh_hbm in ((0, xa_hbm), (1, xb_hbm)):
            pltpu.sync_copy(xh_hbm.at[pl.ds(r0, ROWS_PER_TILE)],
                            acc_sh.at[pl.ds(r0, ROWS_PER_TILE)])

            @pl.when(sid == NS - 1)
            def _():
                pltpu.sync_copy(xh_hbm.at[pl.ds(TAIL_BASE, TAIL_ROWS)],
                                acc_sh.at[pl.ds(TAIL_BASE, TAIL_ROWS)])

            plsc.subcore_barrier()

            def gather(i, b):
                pltpu.async_copy(xh_hbm.at[src_v.at[i]], bufs[b], gsems[b])

            def wait_gather(i, b):
                pltpu.make_async_copy(xh_hbm.at[src_v.at[i]],
                                      bufs[b], gsems[b]).wait()

            def scatter(i, b):
                pltpu.sync_copy(bufs[b], acc_sh.at[dst_v.at[i]], add=True)

            def step(j, b):
                wait_gather(j, b)
                scatter(j, b)
                if isinstance(j, int):
                    nxt = min(j + 3, NCHUNK - 1)
                else:
                    nxt = jnp.minimum(j + 3, NCHUNK - 1)
                gather(nxt, (b + 3) % 4)

            gather(0, 0)
            gather(1, 1)
            gather(2, 2)

            def body(m, _):
                j = 4 * m
                step(j + 0, 0)
                step(j + 1, 1)
                step(j + 2, 2)
                step(j + 3, 3)
                return ()

            lax.fori_loop(0, NCHUNK // 4, body, (), unroll=False)
            for j in range((NCHUNK // 4) * 4, NCHUNK):
                step(j, j % 4)
            for j in (NCHUNK - 3, NCHUNK - 2, NCHUNK - 1):
                wait_gather(NCHUNK - 1, (j % 4 + 3) % 4)

            plsc.subcore_barrier()
            pltpu.sync_copy(acc_sh.at[pl.ds(r0, ROWS_PER_TILE)],
                            out_hbm.at[cid, h, pl.ds(r0, ROWS_PER_TILE)])

            @pl.when(sid == NS - 1)
            def _():
                pltpu.sync_copy(acc_sh.at[pl.ds(TAIL_BASE, TAIL_ROWS)],
                                out_hbm.at[cid, h, pl.ds(TAIL_BASE, TAIL_ROWS)])

    return scatter_kernel(xa, xb, edges)


def _tc_mlp_body(pre_ref, w1t_ref, w2t_ref, g1_ref, b1_ref, g2_ref, b2_ref, out_ref):
    w1t = w1t_ref[0]
    h = jnp.dot(pre_ref[0, 0], w1t[:DH, :], preferred_element_type=jnp.float32)
    h = h + jnp.dot(pre_ref[0, 1], w1t[DH:, :], preferred_element_type=jnp.float32)
    mean = jnp.mean(h, axis=0, keepdims=True)
    var = jnp.mean((h - mean) * (h - mean), axis=0, keepdims=True)
    h = (h - mean) * lax.rsqrt(var + BN_EPS) * g1_ref[0] + b1_ref[0]
    h = jnp.maximum(h, 0.0)
    h = jnp.dot(h, w2t_ref[0], preferred_element_type=jnp.float32)
    mean = jnp.mean(h, axis=0, keepdims=True)
    var = jnp.mean((h - mean) * (h - mean), axis=0, keepdims=True)
    h = (h - mean) * lax.rsqrt(var + BN_EPS) * g2_ref[0] + b2_ref[0]
    h = jnp.maximum(h, 0.0)

    @pl.when(pl.program_id(0) == 0)
    def _():
        out_ref[...] = h

    @pl.when(pl.program_id(0) == 1)
    def _():
        out_ref[...] += h


def _tc_mlp(pre, w1t, w2t, g1, b1, g2, b2):
    rel_spec = pl.BlockSpec((1, 2, N, DH), lambda r: (r, 0, 0, 0))
    w_spec = pl.BlockSpec((1, D, D), lambda r: (r, 0, 0))
    v_spec = pl.BlockSpec((1, 1, D), lambda r: (r, 0, 0))
    return pl.pallas_call(
        _tc_mlp_body,
        grid=(2,),
        in_specs=[rel_spec, w_spec, w_spec, v_spec, v_spec, v_spec, v_spec],
        out_specs=pl.BlockSpec((N, D), lambda r: (0, 0)),
        out_shape=jax.ShapeDtypeStruct((N, D), jnp.float32),
    )(pre, w1t, w2t, g1, b1, g2, b2)


def kernel(x, edge_index_rel0, edge_index_rel1,
           W1_0, W2_0, g1_0, b1_0, g2_0, b2_0,
           W1_1, W2_1, g1_1, b1_1, g2_1, b2_1):
    edges = jnp.stack([edge_index_rel0, edge_index_rel1])
    if PAD:
        edges = edges.reshape(2, 2, NS, EDGES_PER_TILE)
        pad_src = jnp.zeros((2, 1, NS, PAD), jnp.int32)
        pad_dst = jnp.broadcast_to(
            TRASH + (jnp.arange(PAD, dtype=jnp.int32) % 8), (2, 1, NS, PAD))
        pad = jnp.concatenate([pad_src, pad_dst], axis=1)
        edges = jnp.concatenate([edges, pad], axis=-1)
    edges = edges.reshape(2, 2, NS, NCHUNK, CHUNK)
    xa = x[:, :DH]
    xb = x[:, DH:]
    pre = _sc_scatter(xa, xb, edges)

    w1t = jnp.stack([W1_0.T, W1_1.T])
    w2t = jnp.stack([W2_0.T, W2_1.T])
    g1 = jnp.stack([g1_0, g1_1]).reshape(2, 1, D)
    b1 = jnp.stack([b1_0, b1_1]).reshape(2, 1, D)
    g2 = jnp.stack([g2_0, g2_1]).reshape(2, 1, D)
    b2 = jnp.stack([b2_0, b2_1]).reshape(2, 1, D)
    return _tc_mlp(pre, w1t, w2t, g1, b1, g2, b2)


# speedup vs baseline: 2.6852x; 2.6852x over previous
import functools

import jax
import jax.numpy as jnp
from jax import lax
from jax.experimental import pallas as pl
from jax.experimental.pallas import tpu as pltpu
from jax.experimental.pallas import tpu_sc as plsc

N = 10000
E = 320000
D = 128
DH = D // 2
BN_EPS = 1e-5

NC = 2
NS = 16

EDGES_PER_TILE = E // NS
CHUNK = 80
NCHUNK = -(-EDGES_PER_TILE // CHUNK)
PAD = NCHUNK * CHUNK - EDGES_PER_TILE
NBUF = 8
TRASH = N
ACC_ROWS = N + 8
ROWS_PER_TILE = 624
TAIL_ROWS = N - NS * ROWS_PER_TILE
TAIL_BASE = NS * ROWS_PER_TILE


def _sc_scatter(xa, xb, edges):
    mesh = plsc.VectorSubcoreMesh(core_axis_name="c", subcore_axis_name="s")

    @functools.partial(
        pl.kernel,
        mesh=mesh,
        compiler_params=pltpu.CompilerParams(use_tc_tiling_on_sc=False),
        out_type=jax.ShapeDtypeStruct((2, 2, N, DH), jnp.float32),
        scratch_types=[
            pltpu.VMEM((NCHUNK, CHUNK), jnp.int32),
            pltpu.VMEM((NCHUNK, CHUNK), jnp.int32),
        ] + [pltpu.VMEM((CHUNK, DH), jnp.float32)] * NBUF
          + [pltpu.VMEM_SHARED((ACC_ROWS, DH), jnp.float32)]
          + [pltpu.SemaphoreType.DMA] * NBUF,
    )
    def scatter_kernel(xa_hbm, xb_hbm, edges_hbm, out_hbm,
                       src_v, dst_v, *rest):
        bufs = list(rest[:NBUF])
        acc_sh = rest[NBUF]
        gsems = list(rest[NBUF + 1:])
        cid = lax.axis_index("c")
        sid = lax.axis_index("s")

        pltpu.sync_copy(edges_hbm.at[cid, 0, sid], src_v)
        pltpu.sync_copy(edges_hbm.at[cid, 1, sid], dst_v)

        r0 = sid * ROWS_PER_TILE

        for h, xh_hbm in ((0, xa_hbm), (1, xb_hbm)):
            pltpu.sync_copy(xh_hbm.at[pl.ds(r0, ROWS_PER_TILE)],
                            acc_sh.at[pl.ds(r0, ROWS_PER_TILE)])

            @pl.when(sid == NS - 1)
            def _():
                pltpu.sync_copy(xh_hbm.at[pl.ds(TAIL_BASE, TAIL_ROWS)],
                                acc_sh.at[pl.ds(TAIL_BASE, TAIL_ROWS)])

            plsc.subcore_barrier()

            def gather(i, b):
                pltpu.async_copy(xh_hbm.at[src_v.at[i]], bufs[b], gsems[b])

            def wait_gather(i, b):
                pltpu.make_async_copy(xh_hbm.at[src_v.at[i]],
                                      bufs[b], gsems[b]).wait()

            def scatter(i, b):
                pltpu.sync_copy(bufs[b], acc_sh.at[dst_v.at[i]], add=True)

            def step(j, b):
                wait_gather(j, b)
                scatter(j, b)
                if isinstance(j, int):
                    nxt = min(j + NBUF - 1, NCHUNK - 1)
                else:
                    nxt = jnp.minimum(j + NBUF - 1, NCHUNK - 1)
                gather(nxt, (b + NBUF - 1) % NBUF)

            for b in range(NBUF - 1):
                gather(b, b)

            def body(m, _):
                j = NBUF * m
                for b in range(NBUF):
                    step(j + b, b)
                return ()

            lax.fori_loop(0, NCHUNK // NBUF, body, (), unroll=False)
            for j in range((NCHUNK // NBUF) * NBUF, NCHUNK):
                step(j, j % NBUF)
            for j in range(NCHUNK - NBUF + 1, NCHUNK):
                wait_gather(NCHUNK - 1, (j % NBUF + NBUF - 1) % NBUF)

            plsc.subcore_barrier()
            pltpu.sync_copy(acc_sh.at[pl.ds(r0, ROWS_PER_TILE)],
                            out_hbm.at[cid, h, pl.ds(r0, ROWS_PER_TILE)])

            @pl.when(sid == NS - 1)
            def _():
                pltpu.sync_copy(acc_sh.at[pl.ds(TAIL_BASE, TAIL_ROWS)],
                                out_hbm.at[cid, h, pl.ds(TAIL_BASE, TAIL_ROWS)])

    return scatter_kernel(xa, xb, edges)


def _tc_mlp_body(pre_ref, w1t_ref, w2t_ref, g1_ref, b1_ref, g2_ref, b2_ref, out_ref):
    w1t = w1t_ref[0]
    h = jnp.dot(pre_ref[0, 0], w1t[:DH, :], preferred_element_type=jnp.float32)
    h = h + jnp.dot(pre_ref[0, 1], w1t[DH:, :], preferred_element_type=jnp.float32)
    mean = jnp.mean(h, axis=0, keepdims=True)
    var = jnp.mean((h - mean) * (h - mean), axis=0, keepdims=True)
    h = (h - mean) * lax.rsqrt(var + BN_EPS) * g1_ref[0] + b1_ref[0]
    h = jnp.maximum(h, 0.0)
    h = jnp.dot(h, w2t_ref[0], preferred_element_type=jnp.float32)
    mean = jnp.mean(h, axis=0, keepdims=True)
    var = jnp.mean((h - mean) * (h - mean), axis=0, keepdims=True)
    h = (h - mean) * lax.rsqrt(var + BN_EPS) * g2_ref[0] + b2_ref[0]
    h = jnp.maximum(h, 0.0)

    @pl.when(pl.program_id(0) == 0)
    def _():
        out_ref[...] = h

    @pl.when(pl.program_id(0) == 1)
    def _():
        out_ref[...] += h


def _tc_mlp(pre, w1t, w2t, g1, b1, g2, b2):
    rel_spec = pl.BlockSpec((1, 2, N, DH), lambda r: (r, 0, 0, 0))
    w_spec = pl.BlockSpec((1, D, D), lambda r: (r, 0, 0))
    v_spec = pl.BlockSpec((1, 1, D), lambda r: (r, 0, 0))
    return pl.pallas_call(
        _tc_mlp_body,
        grid=(2,),
        in_specs=[rel_spec, w_spec, w_spec, v_spec, v_spec, v_spec, v_spec],
        out_specs=pl.BlockSpec((N, D), lambda r: (0, 0)),
        out_shape=jax.ShapeDtypeStruct((N, D), jnp.float32),
    )(pre, w1t, w2t, g1, b1, g2, b2)


def kernel(x, edge_index_rel0, edge_index_rel1,
           W1_0, W2_0, g1_0, b1_0, g2_0, b2_0,
           W1_1, W2_1, g1_1, b1_1, g2_1, b2_1):
    edges = jnp.stack([edge_index_rel0, edge_index_rel1])
    if PAD:
        edges = edges.reshape(2, 2, NS, EDGES_PER_TILE)
        pad_src = jnp.zeros((2, 1, NS, PAD), jnp.int32)
        pad_dst = jnp.broadcast_to(
            TRASH + (jnp.arange(PAD, dtype=jnp.int32) % 8), (2, 1, NS, PAD))
        pad = jnp.concatenate([pad_src, pad_dst], axis=1)
        edges = jnp.concatenate([edges, pad], axis=-1)
    edges = edges.reshape(2, 2, NS, NCHUNK, CHUNK)
    xa = x[:, :DH]
    xb = x[:, DH:]
    pre = _sc_scatter(xa, xb, edges)

    w1t = jnp.stack([W1_0.T, W1_1.T])
    w2t = jnp.stack([W2_0.T, W2_1.T])
    g1 = jnp.stack([g1_0, g1_1]).reshape(2, 1, D)
    b1 = jnp.stack([b1_0, b1_1]).reshape(2, 1, D)
    g2 = jnp.stack([g2_0, g2_1]).reshape(2, 1, D)
    b2 = jnp.stack([b2_0, b2_1]).reshape(2, 1, D)
    return _tc_mlp(pre, w1t, w2t, g1, b1, g2, b2)
